# baseline (speedup 1.0000x reference)
import jax
import jax.numpy as jnp
from jax.experimental import pallas as pl
from jax.experimental.pallas import tpu as pltpu

C = 224
H = 512
W = 512
NPIX = H * W
K = 64
B = 28


def _halve_axis1(d, fn, rows=8):
    while d.shape[1] > rows:
        h = d.shape[1] // 2
        d = fn(d[:, :h], d[:, h:])
    return d


def _two_sum(a, b):
    s = a + b
    bb = s - a
    e = (a - bb) + (b - (s - bb))
    return s, e


def _dd_add(xh, xl, yh, yl):
    s, e = _two_sum(xh, yh)
    e = e + (xl + yl)
    hi = s + e
    lo = e - (hi - s)
    return hi, lo


def _dd_halve(hi, lo):
    while hi.shape[0] > 1:
        h = hi.shape[0] // 2
        hi, lo = _dd_add(hi[:h], lo[:h], hi[h:], lo[h:])
    return hi, lo


def _dd_tree8(planes):
    p01 = _two_sum(planes[0], planes[1])
    p23 = _two_sum(planes[2], planes[3])
    p45 = _two_sum(planes[4], planes[5])
    p67 = _two_sum(planes[6], planes[7])
    a = _dd_add(*p01, *p23)
    b = _dd_add(*p45, *p67)
    return _dd_add(*a, *b)


def _row_to_col(row):
    L = row.shape[1]
    sub = jax.lax.broadcasted_iota(jnp.int32, (L, L), 0)
    lane = jax.lax.broadcasted_iota(jnp.int32, (L, L), 1)
    d = jnp.where(sub == lane, row, 0.0)
    return jnp.sum(d, axis=1, keepdims=True)


def _stats_kernel(x_ref, o_ref):
    d = x_ref[...]
    h = d.shape[1] // 2
    a = d[:, :h]
    b = d[:, h:]
    s = a + b
    q = a * a + b * b
    mn = jnp.minimum(a, b)
    mx = jnp.maximum(a, b)
    o_ref[:, 0:1, :] = _halve_axis1(s, jnp.add, 1)
    o_ref[:, 1:2, :] = _halve_axis1(q, jnp.add, 1)
    o_ref[:, 2:3, :] = _halve_axis1(mn, jnp.minimum, 1)
    o_ref[:, 3:4, :] = _halve_axis1(mx, jnp.maximum, 1)


def _select_kernel(st_ref, idx_ref, norm_ref):
    st = st_ref[...]
    sT_hi = jnp.transpose(st[:, 0, :])
    ssT_hi = jnp.transpose(st[:, 1, :])
    s_hi, s_lo = _dd_halve(sT_hi, jnp.zeros_like(sT_hi))
    ss_hi, ss_lo = _dd_halve(ssT_hi, jnp.zeros_like(ssT_hi))
    inv_n = 1.0 / NPIX
    t = s_hi * s_hi * inv_n
    t2 = 2.0 * s_hi * s_lo * inv_n
    vr_hi, vr_lo = _dd_add(ss_hi, ss_lo, -t, -t2)
    v_hi = _row_to_col(vr_hi)
    v_lo = _row_to_col(vr_lo)
    sub = jax.lax.broadcasted_iota(jnp.int32, (C, C), 0)
    lane = jax.lax.broadcasted_iota(jnp.int32, (C, C), 1)
    gt = ((vr_hi > v_hi)
          | ((vr_hi == v_hi) & (vr_lo > v_lo))
          | ((vr_hi == v_hi) & (vr_lo == v_lo) & (lane < sub)))
    rank = jnp.sum(gt.astype(jnp.int32), axis=1, keepdims=True)
    rank3 = rank.reshape(C, 1, 1)
    slot = (jax.lax.broadcasted_iota(jnp.int32, (C, 8, 128), 1) * 128
            + jax.lax.broadcasted_iota(jnp.int32, (C, 8, 128), 2))
    band = jax.lax.broadcasted_iota(jnp.int32, (C, 8, 128), 0)
    idx_ref[...] = jnp.sum(jnp.where(rank3 == slot, band, 0), axis=0)
    mn_g = jnp.min(st[:, 2, :])
    mx_g = jnp.max(st[:, 3, :])
    inv = 1.0 / (mx_g - mn_g)
    sub8 = jax.lax.broadcasted_iota(jnp.int32, (8, 128), 0)
    norm_ref[...] = jnp.where(sub8 == 0, mn_g,
                    jnp.where(sub8 == 1, inv, 0.0))


def _gather_kernel(idx_ref, x0, x1, x2, x3, x4, x5, x6, x7, norm_ref,
                   o_ref):
    mn = norm_ref[0, 0]
    inv = norm_ref[1, 0]
    refs = (x0, x1, x2, x3, x4, x5, x6, x7)
    for b in range(8):
        o_ref[0, :, b, :] = (refs[b][0] - mn) * inv


def kernel(x):
    stats = pl.pallas_call(
        _stats_kernel,
        grid=(C // B,),
        in_specs=[pl.BlockSpec((B, H, W), lambda i: (i, 0, 0))],
        out_specs=pl.BlockSpec((B, 4, W), lambda i: (i, 0, 0)),
        out_shape=jax.ShapeDtypeStruct((C, 4, W), jnp.float32),
    )(x)

    idx_mat, norm = pl.pallas_call(
        _select_kernel,
        out_shape=(jax.ShapeDtypeStruct((8, 128), jnp.int32),
                   jax.ShapeDtypeStruct((8, 128), jnp.float32)),
    )(stats)

    idx = idx_mat.reshape(-1)[:K]

    band_specs = [
        pl.BlockSpec((1, H, W),
                     lambda g, idx_ref, b=b: (idx_ref[8 * g + b], 0, 0))
        for b in range(8)
    ]
    z = pl.pallas_call(
        _gather_kernel,
        grid_spec=pltpu.PrefetchScalarGridSpec(
            num_scalar_prefetch=1,
            grid=(8,),
            in_specs=band_specs + [
                pl.BlockSpec((8, 128), lambda g, idx_ref: (0, 0)),
            ],
            out_specs=pl.BlockSpec((1, H, 8, W),
                                   lambda g, idx_ref: (g, 0, 0, 0)),
        ),
        out_shape=jax.ShapeDtypeStruct((8, H, 8, W), jnp.float32),
    )(idx, x, x, x, x, x, x, x, x, norm)

    return jnp.transpose(z, (0, 2, 1, 3)).reshape(K, NPIX)


# speedup vs baseline: 1.0064x; 1.0064x over previous
import jax
import jax.numpy as jnp
from jax.experimental import pallas as pl
from jax.experimental.pallas import tpu as pltpu

C = 224
H = 512
W = 512
NPIX = H * W
K = 64
B = 16


def _halve_axis1(d, fn, rows=8):
    while d.shape[1] > rows:
        h = d.shape[1] // 2
        d = fn(d[:, :h], d[:, h:])
    return d


def _two_sum(a, b):
    s = a + b
    bb = s - a
    e = (a - bb) + (b - (s - bb))
    return s, e


def _dd_add(xh, xl, yh, yl):
    s, e = _two_sum(xh, yh)
    e = e + (xl + yl)
    hi = s + e
    lo = e - (hi - s)
    return hi, lo


def _dd_halve(hi, lo):
    while hi.shape[0] > 1:
        h = hi.shape[0] // 2
        hi, lo = _dd_add(hi[:h], lo[:h], hi[h:], lo[h:])
    return hi, lo


def _dd_tree8(planes):
    p01 = _two_sum(planes[0], planes[1])
    p23 = _two_sum(planes[2], planes[3])
    p45 = _two_sum(planes[4], planes[5])
    p67 = _two_sum(planes[6], planes[7])
    a = _dd_add(*p01, *p23)
    b = _dd_add(*p45, *p67)
    return _dd_add(*a, *b)


def _row_to_col(row):
    L = row.shape[1]
    sub = jax.lax.broadcasted_iota(jnp.int32, (L, L), 0)
    lane = jax.lax.broadcasted_iota(jnp.int32, (L, L), 1)
    d = jnp.where(sub == lane, row, 0.0)
    return jnp.sum(d, axis=1, keepdims=True)


def _stats_kernel(x_ref, o_ref):
    d = x_ref[...]
    h = d.shape[1] // 2
    a = d[:, :h]
    b = d[:, h:]
    s = a + b
    q = a * a + b * b
    mn = jnp.minimum(a, b)
    mx = jnp.maximum(a, b)
    o_ref[:, 0:1, :] = _halve_axis1(s, jnp.add, 1)
    o_ref[:, 1:2, :] = _halve_axis1(q, jnp.add, 1)
    o_ref[:, 2:3, :] = _halve_axis1(mn, jnp.minimum, 1)
    o_ref[:, 3:4, :] = _halve_axis1(mx, jnp.maximum, 1)


def _select_kernel(st_ref, idx_ref, norm_ref):
    st = st_ref[...]
    sT_hi = jnp.transpose(st[:, 0, :])
    ssT_hi = jnp.transpose(st[:, 1, :])
    s_hi, s_lo = _dd_halve(sT_hi, jnp.zeros_like(sT_hi))
    ss_hi, ss_lo = _dd_halve(ssT_hi, jnp.zeros_like(ssT_hi))
    inv_n = 1.0 / NPIX
    t = s_hi * s_hi * inv_n
    t2 = 2.0 * s_hi * s_lo * inv_n
    vr_hi, vr_lo = _dd_add(ss_hi, ss_lo, -t, -t2)
    v_hi = _row_to_col(vr_hi)
    v_lo = _row_to_col(vr_lo)
    sub = jax.lax.broadcasted_iota(jnp.int32, (C, C), 0)
    lane = jax.lax.broadcasted_iota(jnp.int32, (C, C), 1)
    gt = ((vr_hi > v_hi)
          | ((vr_hi == v_hi) & (vr_lo > v_lo))
          | ((vr_hi == v_hi) & (vr_lo == v_lo) & (lane < sub)))
    rank = jnp.sum(gt.astype(jnp.int32), axis=1, keepdims=True)
    rank3 = rank.reshape(C, 1, 1)
    slot = (jax.lax.broadcasted_iota(jnp.int32, (C, 8, 128), 1) * 128
            + jax.lax.broadcasted_iota(jnp.int32, (C, 8, 128), 2))
    band = jax.lax.broadcasted_iota(jnp.int32, (C, 8, 128), 0)
    idx_ref[...] = jnp.sum(jnp.where(rank3 == slot, band, 0), axis=0)
    mn_g = jnp.min(st[:, 2, :])
    mx_g = jnp.max(st[:, 3, :])
    inv = 1.0 / (mx_g - mn_g)
    sub8 = jax.lax.broadcasted_iota(jnp.int32, (8, 128), 0)
    norm_ref[...] = jnp.where(sub8 == 0, mn_g,
                    jnp.where(sub8 == 1, inv, 0.0))


def _gather_kernel(idx_ref, x0, x1, x2, x3, x4, x5, x6, x7, norm_ref,
                   o_ref):
    mn = norm_ref[0, 0]
    inv = norm_ref[1, 0]
    refs = (x0, x1, x2, x3, x4, x5, x6, x7)
    for b in range(8):
        o_ref[0, :, b, :] = (refs[b][0] - mn) * inv


def kernel(x):
    stats = pl.pallas_call(
        _stats_kernel,
        grid=(C // B,),
        in_specs=[pl.BlockSpec((B, H, W), lambda i: (i, 0, 0))],
        out_specs=pl.BlockSpec((B, 4, W), lambda i: (i, 0, 0)),
        out_shape=jax.ShapeDtypeStruct((C, 4, W), jnp.float32),
    )(x)

    idx_mat, norm = pl.pallas_call(
        _select_kernel,
        out_shape=(jax.ShapeDtypeStruct((8, 128), jnp.int32),
                   jax.ShapeDtypeStruct((8, 128), jnp.float32)),
    )(stats)

    idx = idx_mat.reshape(-1)[:K]

    band_specs = [
        pl.BlockSpec((1, H, W),
                     lambda g, idx_ref, b=b: (idx_ref[8 * g + b], 0, 0))
        for b in range(8)
    ]
    z = pl.pallas_call(
        _gather_kernel,
        grid_spec=pltpu.PrefetchScalarGridSpec(
            num_scalar_prefetch=1,
            grid=(8,),
            in_specs=band_specs + [
                pl.BlockSpec((8, 128), lambda g, idx_ref: (0, 0)),
            ],
            out_specs=pl.BlockSpec((1, H, 8, W),
                                   lambda g, idx_ref: (g, 0, 0, 0)),
        ),
        out_shape=jax.ShapeDtypeStruct((8, H, 8, W), jnp.float32),
    )(idx, x, x, x, x, x, x, x, x, norm)

    return jnp.transpose(z, (0, 2, 1, 3)).reshape(K, NPIX)
